# Initial kernel scaffold; baseline (speedup 1.0000x reference)
import jax
import jax.numpy as jnp
from jax.experimental import pallas as pl


def kernel(x, table0, table1, table2, table3):
    raise NotImplementedError("write your pallas kernel here")


# speedup vs baseline: 5.5340x; 5.5340x over previous
import functools

import jax
import jax.numpy as jnp
from jax import lax
from jax.experimental import pallas as pl
from jax.experimental.pallas import tpu as pltpu
from jax.experimental.pallas import tpu_sc as plsc

DIMS = (64, 32, 32, 16)
OFFS = (0, 64, 96, 128)
DTOT = 144
NBLK = 128
BLKS_PER_CHUNK = 5
CHUNK = NBLK * BLKS_PER_CHUNK
GROUPS = CHUNK // 16


def _sc_embed(xt3, table0, table1, table2, table3, n_total):
    info = plsc.get_sparse_core_info()
    nc, ns = info.num_cores, info.num_subcores
    nw = nc * ns
    per_w = n_total // nw
    chunks = per_w // CHUNK
    mesh = plsc.VectorSubcoreMesh(core_axis_name="c", subcore_axis_name="s")

    @functools.partial(
        pl.kernel,
        mesh=mesh,
        compiler_params=pltpu.CompilerParams(
            use_tc_tiling_on_sc=False, needs_layout_passes=False),
        out_type=jax.ShapeDtypeStruct((n_total, DTOT), jnp.float32),
        scratch_types=(
            [pltpu.VMEM((BLKS_PER_CHUNK, NBLK), jnp.int32) for _ in range(4)]
            + [pltpu.VMEM((CHUNK, d), jnp.float32) for d in DIMS]
            + [pltpu.SemaphoreType.DMA]
        ),
    )
    def k(xt_hbm, t0, t1, t2, t3, out_hbm,
          i0, i1, i2, i3, e0, e1, e2, e3, sem):
        tabs = (t0, t1, t2, t3)
        idxs = (i0, i1, i2, i3)
        embs = (e0, e1, e2, e3)
        wid = lax.axis_index("s") * nc + lax.axis_index("c")
        blk0 = wid * (per_w // NBLK)

        def chunk_body(c, carry):
            cb = blk0 + c * BLKS_PER_CHUNK
            row0 = cb * NBLK
            for f in range(4):
                pltpu.sync_copy(xt_hbm.at[f, pl.ds(cb, BLKS_PER_CHUNK), :],
                                idxs[f])
            handles = []
            for f in range(4):
                for j in range(BLKS_PER_CHUNK):
                    handles.append(pltpu.async_copy(
                        tabs[f].at[idxs[f].at[j]],
                        embs[f].at[pl.ds(j * NBLK, NBLK)],
                        sem))
            for h in handles:
                h.wait()

            for f in range(4):
                def mask_body(g, carry2, f=f):
                    j = g // (NBLK // 16)
                    o = (g % (NBLK // 16)) * 16
                    iv = idxs[f][j, pl.ds(o, 16)]
                    rows = j * NBLK + o + lax.iota(jnp.int32, 16)
                    m = iv == 0

                    @pl.when(jnp.any(m))
                    def _():
                        z = jnp.zeros((16,), jnp.float32)
                        for col in range(DIMS[f]):
                            plsc.store_scatter(
                                embs[f],
                                [rows, jnp.full((16,), col, jnp.int32)],
                                z, mask=m)
                    return carry2
                lax.fori_loop(0, GROUPS, mask_body, 0)

            for f in range(4):
                pltpu.sync_copy(
                    embs[f],
                    out_hbm.at[pl.ds(row0, CHUNK), pl.ds(OFFS[f], DIMS[f])])
            return carry

        lax.fori_loop(0, chunks, chunk_body, 0)

    return k(xt3, table0, table1, table2, table3)


def kernel(x, table0, table1, table2, table3):
    b, l, f = x.shape
    n = b * l
    xt3 = x.reshape(n, f).T.reshape(f, n // NBLK, NBLK)
    out = _sc_embed(xt3, table0, table1, table2, table3, n)
    return out.reshape(b, l, DTOT)


# speedup vs baseline: 6.1056x; 1.1033x over previous
import functools

import jax
import jax.numpy as jnp
from jax import lax
from jax.experimental import pallas as pl
from jax.experimental.pallas import tpu as pltpu
from jax.experimental.pallas import tpu_sc as plsc

DIMS = (64, 32, 32, 16)
OFFS = (0, 64, 96, 128)
DTOT = 144
CHUNK = 128
NBUF = 5


def _sc_embed(xt4, table0, table1, table2, table3, n_total):
    info = plsc.get_sparse_core_info()
    nc, ns = info.num_cores, info.num_subcores
    nw = nc * ns
    per_w = n_total // nw
    blks = per_w // CHUNK
    rounds = blks // NBUF
    mesh = plsc.VectorSubcoreMesh(core_axis_name="c", subcore_axis_name="s")

    @functools.partial(
        pl.kernel,
        mesh=mesh,
        compiler_params=pltpu.CompilerParams(
            use_tc_tiling_on_sc=False, needs_layout_passes=False),
        out_type=jax.ShapeDtypeStruct((n_total, DTOT), jnp.float32),
        scratch_types=(
            [pltpu.VMEM((blks, CHUNK), jnp.int32) for _ in range(4)],
            [[pltpu.VMEM((CHUNK, d), jnp.float32) for d in DIMS]
             for _ in range(NBUF)],
            [pltpu.SemaphoreType.DMA for _ in range(NBUF)],
            [pltpu.SemaphoreType.DMA for _ in range(NBUF)],
        ),
    )
    def k(xt_hbm, t0, t1, t2, t3, out_hbm, idxs, embs, gsems, wsems):
        tabs = (t0, t1, t2, t3)
        wid = lax.axis_index("s") * nc + lax.axis_index("c")
        row_base = wid * per_w

        for f in range(4):
            pltpu.sync_copy(xt_hbm.at[f, wid], idxs[f])

        def out_slc(c, f):
            return out_hbm.at[pl.ds(row_base + c * CHUNK, CHUNK),
                              pl.ds(OFFS[f], DIMS[f])]

        def outer(r, carry):
            c0 = r * NBUF
            for b in range(NBUF):
                c = c0 + b
                @pl.when(r > 0)
                def _(b=b, c=c):
                    for f in range(4):
                        pltpu.make_async_copy(
                            embs[b][f], out_slc(c, f), wsems[b]).wait()
                for f in range(4):
                    pltpu.async_copy(
                        tabs[f].at[idxs[f].at[c]], embs[b][f], gsems[b])
            for b in range(NBUF):
                c = c0 + b
                for f in range(4):
                    pltpu.make_async_copy(
                        tabs[f].at[idxs[f].at[c]], embs[b][f], gsems[b]).wait()
                for f in range(4):
                    hit = jnp.zeros((16,), jnp.bool_)
                    for g in range(CHUNK // 16):
                        hit = jnp.logical_or(
                            hit, idxs[f][c, pl.ds(g * 16, 16)] == 0)

                    @pl.when(jnp.any(hit))
                    def _(b=b, c=c, f=f):
                        def zero_group(g, carry2):
                            iv = idxs[f][c, pl.ds(g * 16, 16)]
                            rows = g * 16 + lax.iota(jnp.int32, 16)
                            z = jnp.zeros((16,), jnp.float32)
                            for col in range(DIMS[f]):
                                plsc.store_scatter(
                                    embs[b][f],
                                    [rows, jnp.full((16,), col, jnp.int32)],
                                    z, mask=iv == 0)
                            return carry2
                        lax.fori_loop(0, CHUNK // 16, zero_group, 0)
                for f in range(4):
                    pltpu.async_copy(embs[b][f], out_slc(c, f), wsems[b])
            return carry

        lax.fori_loop(0, rounds, outer, 0)

        for b in range(NBUF):
            for f in range(4):
                pltpu.make_async_copy(
                    embs[b][f], out_slc(b, f), wsems[b]).wait()

    return k(xt4, table0, table1, table2, table3)


def kernel(x, table0, table1, table2, table3):
    b, l, f = x.shape
    n = b * l
    info = plsc.get_sparse_core_info()
    nw = info.num_cores * info.num_subcores
    xt4 = x.reshape(n, f).T.reshape(f, nw, (n // nw) // CHUNK, CHUNK)
    out = _sc_embed(xt4, table0, table1, table2, table3, n)
    return out.reshape(b, l, DTOT)
